# baseline (speedup 1.0000x reference)
import functools

import jax
import jax.numpy as jnp
from jax import lax
from jax.experimental import pallas as pl
from jax.experimental.pallas import tpu as pltpu
from jax.experimental.pallas import tpu_sc as plsc

_RING = 4
_LANES = 16


def kernel(r_idx, kv):
    b, p2, w2, c_kv = kv.shape
    topk = r_idx.shape[2]
    qk = p2 * topk
    blk = w2 * c_kv
    total = b * qk

    nc, ns = 2, 16
    rounds = b // nc
    max_rows = qk // ns + 1

    starts = [12 * s + min(s, 4) for s in range(ns)]

    sub = blk // 128
    kv2 = kv.reshape(b * p2, sub, 128)

    idx2 = jnp.pad(r_idx.reshape(b, qk).astype(jnp.int32), ((0, 0), (0, 4)))
    win = jnp.stack([idx2[:, st:st + max_rows] for st in starts], axis=1)
    win = jnp.pad(win, ((0, 0), (0, 0), (0, _LANES - max_rows)))
    winc = win.reshape(nc, rounds, ns, _LANES)
    idx_pre = jnp.transpose(winc, (1, 2, 0, 3)).reshape(rounds * nc * ns * _LANES)

    mesh = plsc.VectorSubcoreMesh(
        core_axis_name="c", subcore_axis_name="s",
        num_cores=nc, num_subcores=ns,
    )

    @functools.partial(
        pl.kernel,
        out_type=jax.ShapeDtypeStruct((total, sub, 128), kv.dtype),
        mesh=mesh,
        scratch_types=[
            pltpu.VMEM_SHARED((p2, sub, 128), jnp.float32),
            pltpu.VMEM((_LANES,), jnp.int32),
            pltpu.SemaphoreType.DMA,
            pltpu.SemaphoreType.DMA((_RING,)),
        ],
    )
    def gather_rows(kv_hbm, idx_hbm, out_hbm, spmem, idx_v, stage_sem, sems):
        c = lax.axis_index("c")
        s = lax.axis_index("s")
        w = s * nc + c
        base_r = 12 * s + jnp.minimum(s, 4)
        nrows = jnp.where(s < 4, 13, 12)

        for n in range(rounds):
            bt = rounds * c + n

            pltpu.sync_copy(idx_hbm.at[pl.ds((n * nc * ns + w) * _LANES, _LANES)], idx_v)

            @pl.when(s == 0)
            def _(bt=bt):
                pltpu.async_copy(
                    kv_hbm.at[pl.ds(bt * p2, p2)], spmem, stage_sem
                ).wait()

            plsc.subcore_barrier()

            gvec = idx_v[...]

            def row_copy(i, bt=bt, gvec=gvec):
                row = bt * qk + base_r + i
                return pltpu.make_async_copy(
                    spmem.at[pl.ds(gvec[i], 1)],
                    out_hbm.at[pl.ds(row, 1)],
                    sems.at[i % _RING],
                )

            for i in range(max_rows):
                def issue(i=i):
                    if i >= _RING:
                        row_copy(i - _RING).wait()
                    row_copy(i).start()

                pl.when(i < nrows)(issue)

            for i in range(max_rows):
                pl.when(
                    jnp.logical_and(i >= nrows - _RING, i < nrows)
                )(lambda i=i: row_copy(i).wait())

            plsc.subcore_barrier()

    out = gather_rows(kv2, idx_pre)
    return out.reshape(b, p2, topk, w2, c_kv)


# speedup vs baseline: 1.0202x; 1.0202x over previous
import functools

import jax
import jax.numpy as jnp
from jax import lax
from jax.experimental import pallas as pl
from jax.experimental.pallas import tpu as pltpu
from jax.experimental.pallas import tpu_sc as plsc

_CHUNK = 8
_SPLIT = 4
_NBUF = 4


def kernel(r_idx, kv):
    b, p2, w2, c_kv = kv.shape
    topk = r_idx.shape[2]
    total = b * p2 * topk * _SPLIT
    blk = (w2 * c_kv) // _SPLIT

    nc, ns = 2, 16
    nw = nc * ns

    per_w = total // nw
    n_chunks = per_w // _CHUNK

    kv_flat = kv.reshape(b * p2 * _SPLIT, blk)
    g_idx = r_idx + (jnp.arange(b, dtype=r_idx.dtype) * p2)[:, None, None]
    g_idx = g_idx.reshape(total // _SPLIT, 1).astype(jnp.int32)
    g_idx = (_SPLIT * g_idx + jnp.arange(_SPLIT, dtype=jnp.int32)).reshape(total)

    mesh = plsc.VectorSubcoreMesh(
        core_axis_name="c", subcore_axis_name="s",
        num_cores=nc, num_subcores=ns,
    )

    @functools.partial(
        pl.kernel,
        out_type=jax.ShapeDtypeStruct((total, blk), kv.dtype),
        mesh=mesh,
        scratch_types=[
            pltpu.VMEM((per_w,), jnp.int32),
            pltpu.VMEM((_NBUF, _CHUNK, blk), jnp.float32),
            pltpu.SemaphoreType.DMA((_NBUF,)),
            pltpu.SemaphoreType.DMA((_NBUF,)),
        ],
    )
    def gather_rows(kv_hbm, idx_hbm, out_hbm, idx_v, buf, gsem, ssem):
        w = lax.axis_index("s") * nc + lax.axis_index("c")
        base = pl.multiple_of(per_w * w, 8)

        pltpu.sync_copy(idx_hbm.at[pl.ds(base, per_w)], idx_v)

        def gather(t):
            return pltpu.make_async_copy(
                kv_hbm.at[idx_v.at[pl.ds(_CHUNK * t, _CHUNK)]],
                buf.at[t % _NBUF],
                gsem.at[t % _NBUF],
            )

        def scatter(t):
            return pltpu.make_async_copy(
                buf.at[t % _NBUF],
                out_hbm.at[pl.ds(base + _CHUNK * t, _CHUNK)],
                ssem.at[t % _NBUF],
            )

        for c in range(n_chunks + 1):
            if c < n_chunks:
                if c >= _NBUF:
                    scatter(c - _NBUF).wait()
                gather(c).start()
            if c >= 1:
                gather(c - 1).wait()
                scatter(c - 1).start()

        for t in range(n_chunks - _NBUF, n_chunks):
            scatter(t).wait()

    out = gather_rows(kv_flat, g_idx)
    return out.reshape(b, p2, topk, w2, c_kv)
